# Initial kernel scaffold; baseline (speedup 1.0000x reference)
import jax
import jax.numpy as jnp
from jax.experimental import pallas as pl


def kernel(s, v, edge_attr, edge_vec, s_proj_w1, s_proj_b1, s_proj_w2, s_proj_b2, edge_proj_w1, edge_proj_b1, edge_proj_w2, edge_proj_b2, final_s_w, final_s_b, final_v_w, U_w, V_w, s_mlp_w1, s_mlp_b1, s_mlp_w2, s_mlp_b2, edge_index):
    raise NotImplementedError("write your pallas kernel here")


# speedup vs baseline: 5.0003x; 5.0003x over previous
import functools

import jax
import jax.numpy as jnp
from jax.experimental import pallas as pl
from jax.experimental.pallas import tpu as pltpu

RES = 0.1
CLAMP = 100.0


def _silu(x):
    return x * jax.nn.sigmoid(x)


def _node_pre_body(s_ref, w1a_ref, w1b_ref, a_ref, b_ref):
    s = s_ref[...]
    a_ref[...] = jnp.dot(s, w1a_ref[...], preferred_element_type=jnp.float32)
    b_ref[...] = jnp.dot(s, w1b_ref[...], preferred_element_type=jnp.float32)


def _node_pre(s, w1a_t, w1b_t, bn):
    n, h = s.shape
    grid = (n // bn,)
    return pl.pallas_call(
        _node_pre_body,
        grid=grid,
        in_specs=[
            pl.BlockSpec((bn, h), lambda i: (i, 0)),
            pl.BlockSpec((h, h), lambda i: (0, 0)),
            pl.BlockSpec((h, h), lambda i: (0, 0)),
        ],
        out_specs=[
            pl.BlockSpec((bn, h), lambda i: (i, 0)),
            pl.BlockSpec((bn, h), lambda i: (i, 0)),
        ],
        out_shape=[
            jax.ShapeDtypeStruct((n, h), jnp.float32),
            jax.ShapeDtypeStruct((n, h), jnp.float32),
        ],
    )(s, w1a_t, w1b_t)


def _edge_body(pre_ref, ea_ref, vj0_ref, vj1_ref, vj2_ref, ev_ref,
               b1_ref, w2t_ref, b2_ref,
               ew1t_ref, eb1_ref, ew2t_ref, eb2_ref,
               ds_ref, dv0_ref, dv1_ref, dv2_ref):
    h = pre_ref.shape[1]
    h1 = _silu(pre_ref[...] + b1_ref[...])
    phis = jnp.dot(h1, w2t_ref[...], preferred_element_type=jnp.float32) + b2_ref[...]
    he = _silu(jnp.dot(ea_ref[...], ew1t_ref[...],
                       preferred_element_type=jnp.float32) + eb1_ref[...])
    phie = jnp.dot(he, ew2t_ref[...], preferred_element_type=jnp.float32) + eb2_ref[...]
    phi = phis * phie
    phi1 = phi[:, :h]
    phi2 = phi[:, h:2 * h]
    phi3 = phi[:, 2 * h:]
    ev = ev_ref[...]
    ds_ref[...] = phi1
    dv0_ref[...] = phi2 * vj0_ref[...] + phi3 * ev[:, 0:1]
    dv1_ref[...] = phi2 * vj1_ref[...] + phi3 * ev[:, 1:2]
    dv2_ref[...] = phi2 * vj2_ref[...] + phi3 * ev[:, 2:3]


def _edge_stage(pre, ea, vj0, vj1, vj2, ev,
                b1, w2t, b2, ew1t, eb1, ew2t, eb2, be):
    e, h = pre.shape
    ed = ea.shape[1]
    grid = (e // be,)
    edge_block = lambda i: (i, 0)
    full = lambda i: (0, 0)
    return pl.pallas_call(
        _edge_body,
        grid=grid,
        in_specs=[
            pl.BlockSpec((be, h), edge_block),
            pl.BlockSpec((be, ed), edge_block),
            pl.BlockSpec((be, h), edge_block),
            pl.BlockSpec((be, h), edge_block),
            pl.BlockSpec((be, h), edge_block),
            pl.BlockSpec((be, 3), edge_block),
            pl.BlockSpec((1, h), full),
            pl.BlockSpec((h, 3 * h), full),
            pl.BlockSpec((1, 3 * h), full),
            pl.BlockSpec((ed, h), full),
            pl.BlockSpec((1, h), full),
            pl.BlockSpec((h, 3 * h), full),
            pl.BlockSpec((1, 3 * h), full),
        ],
        out_specs=[pl.BlockSpec((be, h), edge_block)] * 4,
        out_shape=[jax.ShapeDtypeStruct((e, h), jnp.float32)] * 4,
    )(pre, ea, vj0, vj1, vj2, ev, b1, w2t, b2, ew1t, eb1, ew2t, eb2)


def _node_update_body(s_ref, v0_ref, v1_ref, v2_ref,
                      ds_ref, dv0_ref, dv1_ref, dv2_ref,
                      fst_ref, fsb_ref, fvt_ref,
                      ut_ref, vt_ref,
                      m1at_ref, m1bt_ref, m1b_ref, m2t_ref, m2b_ref,
                      so_ref, vo0_ref, vo1_ref, vo2_ref):
    h = s_ref.shape[1]
    ds = jnp.dot(ds_ref[...], fst_ref[...], preferred_element_type=jnp.float32) + fsb_ref[...]
    s1 = s_ref[...] + RES * ds
    fvt = fvt_ref[...]
    v0 = v0_ref[...] + RES * jnp.dot(dv0_ref[...], fvt, preferred_element_type=jnp.float32)
    v1 = v1_ref[...] + RES * jnp.dot(dv1_ref[...], fvt, preferred_element_type=jnp.float32)
    v2 = v2_ref[...] + RES * jnp.dot(dv2_ref[...], fvt, preferred_element_type=jnp.float32)
    ut = ut_ref[...]
    vt = vt_ref[...]
    uv0 = jnp.dot(v0, ut, preferred_element_type=jnp.float32)
    uv1 = jnp.dot(v1, ut, preferred_element_type=jnp.float32)
    uv2 = jnp.dot(v2, ut, preferred_element_type=jnp.float32)
    vv0 = jnp.dot(v0, vt, preferred_element_type=jnp.float32)
    vv1 = jnp.dot(v1, vt, preferred_element_type=jnp.float32)
    vv2 = jnp.dot(v2, vt, preferred_element_type=jnp.float32)
    vnorm = jnp.sqrt(vv0 * vv0 + vv1 * vv1 + vv2 * vv2)
    m1 = (jnp.dot(s1, m1at_ref[...], preferred_element_type=jnp.float32)
          + jnp.dot(vnorm, m1bt_ref[...], preferred_element_type=jnp.float32)
          + m1b_ref[...])
    a = jnp.dot(_silu(m1), m2t_ref[...], preferred_element_type=jnp.float32) + m2b_ref[...]
    a1 = a[:, :h]
    a2 = a[:, h:2 * h]
    a3 = a[:, 2 * h:]
    so_ref[...] = jnp.clip(s1 + RES * (a1 + a2 * vnorm), -CLAMP, CLAMP)
    vo0_ref[...] = jnp.clip(v0 + RES * (a3 * uv0), -CLAMP, CLAMP)
    vo1_ref[...] = jnp.clip(v1 + RES * (a3 * uv1), -CLAMP, CLAMP)
    vo2_ref[...] = jnp.clip(v2 + RES * (a3 * uv2), -CLAMP, CLAMP)


def _node_update(s, v0, v1, v2, ds, dv0, dv1, dv2,
                 fst, fsb, fvt, ut, vt, m1at, m1bt, m1b, m2t, m2b, bn):
    n, h = s.shape
    grid = (n // bn,)
    nb = lambda i: (i, 0)
    full = lambda i: (0, 0)
    return pl.pallas_call(
        _node_update_body,
        grid=grid,
        in_specs=[pl.BlockSpec((bn, h), nb)] * 8 + [
            pl.BlockSpec((h, h), full),
            pl.BlockSpec((1, h), full),
            pl.BlockSpec((h, h), full),
            pl.BlockSpec((h, h), full),
            pl.BlockSpec((h, h), full),
            pl.BlockSpec((h, h), full),
            pl.BlockSpec((h, h), full),
            pl.BlockSpec((1, h), full),
            pl.BlockSpec((h, 3 * h), full),
            pl.BlockSpec((1, 3 * h), full),
        ],
        out_specs=[pl.BlockSpec((bn, h), nb)] * 4,
        out_shape=[jax.ShapeDtypeStruct((n, h), jnp.float32)] * 4,
    )(s, v0, v1, v2, ds, dv0, dv1, dv2,
      fst, fsb, fvt, ut, vt, m1at, m1bt, m1b, m2t, m2b)


def kernel(s, v, edge_attr, edge_vec,
           s_proj_w1, s_proj_b1, s_proj_w2, s_proj_b2,
           edge_proj_w1, edge_proj_b1, edge_proj_w2, edge_proj_b2,
           final_s_w, final_s_b, final_v_w,
           U_w, V_w,
           s_mlp_w1, s_mlp_b1, s_mlp_w2, s_mlp_b2,
           edge_index):
    n, h = s.shape
    e = edge_attr.shape[0]
    src = edge_index[0]
    dst = edge_index[1]

    w1a_t = s_proj_w1[:, :h].T
    w1b_t = s_proj_w1[:, h:].T
    a_tab, b_tab = _node_pre(s, w1a_t, w1b_t, bn=1000)

    pre = jnp.take(a_tab, dst, axis=0) + jnp.take(b_tab, src, axis=0)
    v_t = jnp.transpose(v, (2, 0, 1))
    vj0 = jnp.take(v_t[0], src, axis=0)
    vj1 = jnp.take(v_t[1], src, axis=0)
    vj2 = jnp.take(v_t[2], src, axis=0)

    ds_msg, dvm0, dvm1, dvm2 = _edge_stage(
        pre, edge_attr, vj0, vj1, vj2, edge_vec,
        s_proj_b1.reshape(1, h), s_proj_w2.T, s_proj_b2.reshape(1, 3 * h),
        edge_proj_w1.T, edge_proj_b1.reshape(1, h),
        edge_proj_w2.T, edge_proj_b2.reshape(1, 3 * h), be=1000)

    ds = jax.ops.segment_sum(ds_msg, dst, num_segments=n)
    dv0 = jax.ops.segment_sum(dvm0, dst, num_segments=n)
    dv1 = jax.ops.segment_sum(dvm1, dst, num_segments=n)
    dv2 = jax.ops.segment_sum(dvm2, dst, num_segments=n)

    m1a_t = s_mlp_w1[:, :h].T
    m1b_t = s_mlp_w1[:, h:].T
    s_out, vo0, vo1, vo2 = _node_update(
        s, v_t[0], v_t[1], v_t[2], ds, dv0, dv1, dv2,
        final_s_w.T, final_s_b.reshape(1, h), final_v_w.T,
        U_w.T, V_w.T, m1a_t, m1b_t, s_mlp_b1.reshape(1, h),
        s_mlp_w2.T, s_mlp_b2.reshape(1, 3 * h), bn=1000)

    v_out = jnp.stack([vo0, vo1, vo2], axis=-1)
    return (s_out, v_out)


# speedup vs baseline: 7.4520x; 1.4903x over previous
import functools

import jax
import jax.numpy as jnp
from jax import lax
from jax.experimental import pallas as pl
from jax.experimental.pallas import tpu as pltpu
from jax.experimental.pallas import tpu_sc as plsc

RES = 0.1
CLAMP = 100.0


def _silu(x):
    return x * jax.nn.sigmoid(x)


def _node_pre_body(s_ref, w1a_ref, w1b_ref, a_ref, b_ref):
    s = s_ref[...]
    a_ref[...] = jnp.dot(s, w1a_ref[...], preferred_element_type=jnp.float32)
    b_ref[...] = jnp.dot(s, w1b_ref[...], preferred_element_type=jnp.float32)


def _node_pre(s, w1a_t, w1b_t, bn):
    n, h = s.shape
    grid = (n // bn,)
    return pl.pallas_call(
        _node_pre_body,
        grid=grid,
        in_specs=[
            pl.BlockSpec((bn, h), lambda i: (i, 0)),
            pl.BlockSpec((h, h), lambda i: (0, 0)),
            pl.BlockSpec((h, h), lambda i: (0, 0)),
        ],
        out_specs=[
            pl.BlockSpec((bn, h), lambda i: (i, 0)),
            pl.BlockSpec((bn, h), lambda i: (i, 0)),
        ],
        out_shape=[
            jax.ShapeDtypeStruct((n, h), jnp.float32),
            jax.ShapeDtypeStruct((n, h), jnp.float32),
        ],
    )(s, w1a_t, w1b_t)


def _edge_body(pre_ref, ea_ref, vj0_ref, vj1_ref, vj2_ref, ev_ref,
               b1_ref, w2t_ref, b2_ref,
               ew1t_ref, eb1_ref, ew2t_ref, eb2_ref,
               ds_ref, dv0_ref, dv1_ref, dv2_ref):
    h = pre_ref.shape[1]
    h1 = _silu(pre_ref[...] + b1_ref[...])
    phis = jnp.dot(h1, w2t_ref[...], preferred_element_type=jnp.float32) + b2_ref[...]
    he = _silu(jnp.dot(ea_ref[...], ew1t_ref[...],
                       preferred_element_type=jnp.float32) + eb1_ref[...])
    phie = jnp.dot(he, ew2t_ref[...], preferred_element_type=jnp.float32) + eb2_ref[...]
    phi = phis * phie
    phi1 = phi[:, :h]
    phi2 = phi[:, h:2 * h]
    phi3 = phi[:, 2 * h:]
    ev = ev_ref[...]
    ds_ref[...] = phi1
    dv0_ref[...] = phi2 * vj0_ref[...] + phi3 * ev[:, 0:1]
    dv1_ref[...] = phi2 * vj1_ref[...] + phi3 * ev[:, 1:2]
    dv2_ref[...] = phi2 * vj2_ref[...] + phi3 * ev[:, 2:3]


def _edge_stage(pre, ea, vj0, vj1, vj2, ev,
                b1, w2t, b2, ew1t, eb1, ew2t, eb2, be):
    e, h = pre.shape
    ed = ea.shape[1]
    grid = (e // be,)
    edge_block = lambda i: (i, 0)
    full = lambda i: (0, 0)
    return pl.pallas_call(
        _edge_body,
        grid=grid,
        in_specs=[
            pl.BlockSpec((be, h), edge_block),
            pl.BlockSpec((be, ed), edge_block),
            pl.BlockSpec((be, h), edge_block),
            pl.BlockSpec((be, h), edge_block),
            pl.BlockSpec((be, h), edge_block),
            pl.BlockSpec((be, 3), edge_block),
            pl.BlockSpec((1, h), full),
            pl.BlockSpec((h, 3 * h), full),
            pl.BlockSpec((1, 3 * h), full),
            pl.BlockSpec((ed, h), full),
            pl.BlockSpec((1, h), full),
            pl.BlockSpec((h, 3 * h), full),
            pl.BlockSpec((1, 3 * h), full),
        ],
        out_specs=[pl.BlockSpec((be, h), edge_block)] * 4,
        out_shape=[jax.ShapeDtypeStruct((e, h), jnp.float32)] * 4,
    )(pre, ea, vj0, vj1, vj2, ev, b1, w2t, b2, ew1t, eb1, ew2t, eb2)


def _node_update_body(s_ref, v0_ref, v1_ref, v2_ref,
                      ds_ref, dv0_ref, dv1_ref, dv2_ref,
                      fst_ref, fsb_ref, fvt_ref,
                      ut_ref, vt_ref,
                      m1at_ref, m1bt_ref, m1b_ref, m2t_ref, m2b_ref,
                      so_ref, vo0_ref, vo1_ref, vo2_ref):
    h = s_ref.shape[1]
    ds = jnp.dot(ds_ref[...], fst_ref[...], preferred_element_type=jnp.float32) + fsb_ref[...]
    s1 = s_ref[...] + RES * ds
    fvt = fvt_ref[...]
    v0 = v0_ref[...] + RES * jnp.dot(dv0_ref[...], fvt, preferred_element_type=jnp.float32)
    v1 = v1_ref[...] + RES * jnp.dot(dv1_ref[...], fvt, preferred_element_type=jnp.float32)
    v2 = v2_ref[...] + RES * jnp.dot(dv2_ref[...], fvt, preferred_element_type=jnp.float32)
    ut = ut_ref[...]
    vt = vt_ref[...]
    uv0 = jnp.dot(v0, ut, preferred_element_type=jnp.float32)
    uv1 = jnp.dot(v1, ut, preferred_element_type=jnp.float32)
    uv2 = jnp.dot(v2, ut, preferred_element_type=jnp.float32)
    vv0 = jnp.dot(v0, vt, preferred_element_type=jnp.float32)
    vv1 = jnp.dot(v1, vt, preferred_element_type=jnp.float32)
    vv2 = jnp.dot(v2, vt, preferred_element_type=jnp.float32)
    vnorm = jnp.sqrt(vv0 * vv0 + vv1 * vv1 + vv2 * vv2)
    m1 = (jnp.dot(s1, m1at_ref[...], preferred_element_type=jnp.float32)
          + jnp.dot(vnorm, m1bt_ref[...], preferred_element_type=jnp.float32)
          + m1b_ref[...])
    a = jnp.dot(_silu(m1), m2t_ref[...], preferred_element_type=jnp.float32) + m2b_ref[...]
    a1 = a[:, :h]
    a2 = a[:, h:2 * h]
    a3 = a[:, 2 * h:]
    so_ref[...] = jnp.clip(s1 + RES * (a1 + a2 * vnorm), -CLAMP, CLAMP)
    vo0_ref[...] = jnp.clip(v0 + RES * (a3 * uv0), -CLAMP, CLAMP)
    vo1_ref[...] = jnp.clip(v1 + RES * (a3 * uv1), -CLAMP, CLAMP)
    vo2_ref[...] = jnp.clip(v2 + RES * (a3 * uv2), -CLAMP, CLAMP)


def _node_update(s, v0, v1, v2, ds, dv0, dv1, dv2,
                 fst, fsb, fvt, ut, vt, m1at, m1bt, m1b, m2t, m2b, bn):
    n, h = s.shape
    grid = (n // bn,)
    nb = lambda i: (i, 0)
    full = lambda i: (0, 0)
    return pl.pallas_call(
        _node_update_body,
        grid=grid,
        in_specs=[pl.BlockSpec((bn, h), nb)] * 8 + [
            pl.BlockSpec((h, h), full),
            pl.BlockSpec((1, h), full),
            pl.BlockSpec((h, h), full),
            pl.BlockSpec((h, h), full),
            pl.BlockSpec((h, h), full),
            pl.BlockSpec((h, h), full),
            pl.BlockSpec((h, h), full),
            pl.BlockSpec((1, h), full),
            pl.BlockSpec((h, 3 * h), full),
            pl.BlockSpec((1, 3 * h), full),
        ],
        out_specs=[pl.BlockSpec((bn, h), nb)] * 4,
        out_shape=[jax.ShapeDtypeStruct((n, h), jnp.float32)] * 4,
    )(s, v0, v1, v2, ds, dv0, dv1, dv2,
      fst, fsb, fvt, ut, vt, m1at, m1bt, m1b, m2t, m2b)


def _make_sc_scatter_pair(e, n, h, w):
    mesh = plsc.VectorSubcoreMesh(core_axis_name="c", subcore_axis_name="s")
    nsub = 16
    rpt = (n // nsub) // 8 * 8
    tail = n - nsub * rpt
    edges_per_tile = e // nsub
    nwin = edges_per_tile // w

    @functools.partial(
        pl.kernel,
        mesh=mesh,
        out_type=[jax.ShapeDtypeStruct((n, h), jnp.float32)] * 2,
        scratch_types=[
            pltpu.VMEM_SHARED((n, h), jnp.float32),
            pltpu.VMEM((w,), jnp.int32),
            pltpu.VMEM((w, h), jnp.float32),
        ],
    )
    def k(upd_a, upd_b, dst_hbm, zeros_hbm, out_a, out_b, acc, idx_v, buf_v):
        cid = lax.axis_index("c")
        sid = lax.axis_index("s")
        r0 = sid * rpt

        def rows_copy(src, dst_ref):
            pltpu.sync_copy(src.at[pl.ds(r0, rpt)], dst_ref.at[pl.ds(r0, rpt)])
            if tail:
                @pl.when(sid == 0)
                def _():
                    pltpu.sync_copy(src.at[pl.ds(nsub * rpt, tail)],
                                    dst_ref.at[pl.ds(nsub * rpt, tail)])

        rows_copy(zeros_hbm, acc)
        plsc.subcore_barrier()

        def run(upd_hbm):
            def body(wi, carry):
                base = sid * edges_per_tile + wi * w
                pltpu.sync_copy(dst_hbm.at[pl.ds(base, w)], idx_v)
                pltpu.sync_copy(upd_hbm.at[pl.ds(base, w)], buf_v)
                pltpu.sync_copy(buf_v, acc.at[idx_v], add=True)
                return carry
            lax.fori_loop(0, nwin, body, 0)

        @pl.when(cid == 0)
        def _():
            run(upd_a)

        @pl.when(cid == 1)
        def _():
            run(upd_b)

        plsc.subcore_barrier()

        @pl.when(cid == 0)
        def _():
            rows_copy(acc, out_a)

        @pl.when(cid == 1)
        def _():
            rows_copy(acc, out_b)

    return k


def kernel(s, v, edge_attr, edge_vec,
           s_proj_w1, s_proj_b1, s_proj_w2, s_proj_b2,
           edge_proj_w1, edge_proj_b1, edge_proj_w2, edge_proj_b2,
           final_s_w, final_s_b, final_v_w,
           U_w, V_w,
           s_mlp_w1, s_mlp_b1, s_mlp_w2, s_mlp_b2,
           edge_index):
    n, h = s.shape
    e = edge_attr.shape[0]
    src = edge_index[0]
    dst = edge_index[1]

    w1a_t = s_proj_w1[:, :h].T
    w1b_t = s_proj_w1[:, h:].T
    a_tab, b_tab = _node_pre(s, w1a_t, w1b_t, bn=1000)

    pre = jnp.take(a_tab, dst, axis=0) + jnp.take(b_tab, src, axis=0)
    v_t = jnp.transpose(v, (2, 0, 1))
    vj0 = jnp.take(v_t[0], src, axis=0)
    vj1 = jnp.take(v_t[1], src, axis=0)
    vj2 = jnp.take(v_t[2], src, axis=0)

    ds_msg, dvm0, dvm1, dvm2 = _edge_stage(
        pre, edge_attr, vj0, vj1, vj2, edge_vec,
        s_proj_b1.reshape(1, h), s_proj_w2.T, s_proj_b2.reshape(1, 3 * h),
        edge_proj_w1.T, edge_proj_b1.reshape(1, h),
        edge_proj_w2.T, edge_proj_b2.reshape(1, 3 * h), be=1000)

    scatter_pair = _make_sc_scatter_pair(e, n, h, w=200)
    zeros_nh = jnp.zeros((n, h), jnp.float32)
    ds, dv0 = scatter_pair(ds_msg, dvm0, dst, zeros_nh)
    dv1, dv2 = scatter_pair(dvm1, dvm2, dst, zeros_nh)

    m1a_t = s_mlp_w1[:, :h].T
    m1b_t = s_mlp_w1[:, h:].T
    s_out, vo0, vo1, vo2 = _node_update(
        s, v_t[0], v_t[1], v_t[2], ds, dv0, dv1, dv2,
        final_s_w.T, final_s_b.reshape(1, h), final_v_w.T,
        U_w.T, V_w.T, m1a_t, m1b_t, s_mlp_b1.reshape(1, h),
        s_mlp_w2.T, s_mlp_b2.reshape(1, 3 * h), bn=1000)

    v_out = jnp.stack([vo0, vo1, vo2], axis=-1)
    return (s_out, v_out)


# speedup vs baseline: 17.6274x; 2.3655x over previous
import functools

import jax
import jax.numpy as jnp
from jax import lax
from jax.experimental import pallas as pl
from jax.experimental.pallas import tpu as pltpu
from jax.experimental.pallas import tpu_sc as plsc

RES = 0.1
CLAMP = 100.0


def _silu(x):
    return x * jax.nn.sigmoid(x)


def _node_pre_body(s_ref, w1a_ref, w1b_ref, a_ref, b_ref):
    s = s_ref[...]
    a_ref[...] = jnp.dot(s, w1a_ref[...], preferred_element_type=jnp.float32)
    b_ref[...] = jnp.dot(s, w1b_ref[...], preferred_element_type=jnp.float32)


def _node_pre(s, w1a_t, w1b_t, bn):
    n, h = s.shape
    grid = (n // bn,)
    return pl.pallas_call(
        _node_pre_body,
        grid=grid,
        in_specs=[
            pl.BlockSpec((bn, h), lambda i: (i, 0)),
            pl.BlockSpec((h, h), lambda i: (0, 0)),
            pl.BlockSpec((h, h), lambda i: (0, 0)),
        ],
        out_specs=[
            pl.BlockSpec((bn, h), lambda i: (i, 0)),
            pl.BlockSpec((bn, h), lambda i: (i, 0)),
        ],
        out_shape=[
            jax.ShapeDtypeStruct((n, h), jnp.float32),
            jax.ShapeDtypeStruct((n, h), jnp.float32),
        ],
    )(s, w1a_t, w1b_t)


def _edge_body(pre_ref, ea_ref, vj0_ref, vj1_ref, vj2_ref, ev_ref,
               b1_ref, w2t_ref, b2_ref,
               ew1t_ref, eb1_ref, ew2t_ref, eb2_ref,
               ds_ref, dv0_ref, dv1_ref, dv2_ref):
    h = pre_ref.shape[1]
    h1 = _silu(pre_ref[...] + b1_ref[...])
    phis = jnp.dot(h1, w2t_ref[...], preferred_element_type=jnp.float32) + b2_ref[...]
    he = _silu(jnp.dot(ea_ref[...], ew1t_ref[...],
                       preferred_element_type=jnp.float32) + eb1_ref[...])
    phie = jnp.dot(he, ew2t_ref[...], preferred_element_type=jnp.float32) + eb2_ref[...]
    phi = phis * phie
    phi1 = phi[:, :h]
    phi2 = phi[:, h:2 * h]
    phi3 = phi[:, 2 * h:]
    ev = ev_ref[...]
    ds_ref[...] = phi1
    dv0_ref[...] = phi2 * vj0_ref[...] + phi3 * ev[:, 0:1]
    dv1_ref[...] = phi2 * vj1_ref[...] + phi3 * ev[:, 1:2]
    dv2_ref[...] = phi2 * vj2_ref[...] + phi3 * ev[:, 2:3]


def _edge_stage(pre, ea, vj0, vj1, vj2, ev,
                b1, w2t, b2, ew1t, eb1, ew2t, eb2, be):
    e, h = pre.shape
    ed = ea.shape[1]
    grid = (e // be,)
    edge_block = lambda i: (i, 0)
    full = lambda i: (0, 0)
    return pl.pallas_call(
        _edge_body,
        grid=grid,
        in_specs=[
            pl.BlockSpec((be, h), edge_block),
            pl.BlockSpec((be, ed), edge_block),
            pl.BlockSpec((be, h), edge_block),
            pl.BlockSpec((be, h), edge_block),
            pl.BlockSpec((be, h), edge_block),
            pl.BlockSpec((be, 3), edge_block),
            pl.BlockSpec((1, h), full),
            pl.BlockSpec((h, 3 * h), full),
            pl.BlockSpec((1, 3 * h), full),
            pl.BlockSpec((ed, h), full),
            pl.BlockSpec((1, h), full),
            pl.BlockSpec((h, 3 * h), full),
            pl.BlockSpec((1, 3 * h), full),
        ],
        out_specs=[pl.BlockSpec((be, h), edge_block)] * 4,
        out_shape=[jax.ShapeDtypeStruct((e, h), jnp.float32)] * 4,
    )(pre, ea, vj0, vj1, vj2, ev, b1, w2t, b2, ew1t, eb1, ew2t, eb2)


def _node_update_body(s_ref, v0_ref, v1_ref, v2_ref,
                      ds_ref, dv0_ref, dv1_ref, dv2_ref,
                      fst_ref, fsb_ref, fvt_ref,
                      ut_ref, vt_ref,
                      m1at_ref, m1bt_ref, m1b_ref, m2t_ref, m2b_ref,
                      so_ref, vo0_ref, vo1_ref, vo2_ref):
    h = s_ref.shape[1]
    ds = jnp.dot(ds_ref[...], fst_ref[...], preferred_element_type=jnp.float32) + fsb_ref[...]
    s1 = s_ref[...] + RES * ds
    fvt = fvt_ref[...]
    v0 = v0_ref[...] + RES * jnp.dot(dv0_ref[...], fvt, preferred_element_type=jnp.float32)
    v1 = v1_ref[...] + RES * jnp.dot(dv1_ref[...], fvt, preferred_element_type=jnp.float32)
    v2 = v2_ref[...] + RES * jnp.dot(dv2_ref[...], fvt, preferred_element_type=jnp.float32)
    ut = ut_ref[...]
    vt = vt_ref[...]
    uv0 = jnp.dot(v0, ut, preferred_element_type=jnp.float32)
    uv1 = jnp.dot(v1, ut, preferred_element_type=jnp.float32)
    uv2 = jnp.dot(v2, ut, preferred_element_type=jnp.float32)
    vv0 = jnp.dot(v0, vt, preferred_element_type=jnp.float32)
    vv1 = jnp.dot(v1, vt, preferred_element_type=jnp.float32)
    vv2 = jnp.dot(v2, vt, preferred_element_type=jnp.float32)
    vnorm = jnp.sqrt(vv0 * vv0 + vv1 * vv1 + vv2 * vv2)
    m1 = (jnp.dot(s1, m1at_ref[...], preferred_element_type=jnp.float32)
          + jnp.dot(vnorm, m1bt_ref[...], preferred_element_type=jnp.float32)
          + m1b_ref[...])
    a = jnp.dot(_silu(m1), m2t_ref[...], preferred_element_type=jnp.float32) + m2b_ref[...]
    a1 = a[:, :h]
    a2 = a[:, h:2 * h]
    a3 = a[:, 2 * h:]
    so_ref[...] = jnp.clip(s1 + RES * (a1 + a2 * vnorm), -CLAMP, CLAMP)
    vo0_ref[...] = jnp.clip(v0 + RES * (a3 * uv0), -CLAMP, CLAMP)
    vo1_ref[...] = jnp.clip(v1 + RES * (a3 * uv1), -CLAMP, CLAMP)
    vo2_ref[...] = jnp.clip(v2 + RES * (a3 * uv2), -CLAMP, CLAMP)


def _node_update(s, v0, v1, v2, ds, dv0, dv1, dv2,
                 fst, fsb, fvt, ut, vt, m1at, m1bt, m1b, m2t, m2b, bn):
    n, h = s.shape
    grid = (n // bn,)
    nb = lambda i: (i, 0)
    full = lambda i: (0, 0)
    return pl.pallas_call(
        _node_update_body,
        grid=grid,
        in_specs=[pl.BlockSpec((bn, h), nb)] * 8 + [
            pl.BlockSpec((h, h), full),
            pl.BlockSpec((1, h), full),
            pl.BlockSpec((h, h), full),
            pl.BlockSpec((h, h), full),
            pl.BlockSpec((h, h), full),
            pl.BlockSpec((h, h), full),
            pl.BlockSpec((h, h), full),
            pl.BlockSpec((1, h), full),
            pl.BlockSpec((h, 3 * h), full),
            pl.BlockSpec((1, 3 * h), full),
        ],
        out_specs=[pl.BlockSpec((bn, h), nb)] * 4,
        out_shape=[jax.ShapeDtypeStruct((n, h), jnp.float32)] * 4,
    )(s, v0, v1, v2, ds, dv0, dv1, dv2,
      fst, fsb, fvt, ut, vt, m1at, m1bt, m1b, m2t, m2b)


def _make_sc_scatter_pair(e, n, h, w):
    mesh = plsc.VectorSubcoreMesh(core_axis_name="c", subcore_axis_name="s")
    nsub = 16
    rpt = (n // nsub) // 8 * 8
    tail = n - nsub * rpt
    edges_per_tile = e // nsub
    nwin = edges_per_tile // w

    @functools.partial(
        pl.kernel,
        mesh=mesh,
        out_type=[jax.ShapeDtypeStruct((n, h), jnp.float32)] * 2,
        scratch_types=[
            pltpu.VMEM_SHARED((n, h), jnp.float32),
            pltpu.VMEM((w,), jnp.int32),
            pltpu.VMEM((w, h), jnp.float32),
        ],
    )
    def k(upd_a, upd_b, dst_hbm, zeros_hbm, out_a, out_b, acc, idx_v, buf_v):
        cid = lax.axis_index("c")
        sid = lax.axis_index("s")
        r0 = sid * rpt

        def rows_copy(src, dst_ref):
            pltpu.sync_copy(src.at[pl.ds(r0, rpt)], dst_ref.at[pl.ds(r0, rpt)])
            if tail:
                @pl.when(sid == 0)
                def _():
                    pltpu.sync_copy(src.at[pl.ds(nsub * rpt, tail)],
                                    dst_ref.at[pl.ds(nsub * rpt, tail)])

        rows_copy(zeros_hbm, acc)
        plsc.subcore_barrier()

        def run(upd_hbm):
            def body(wi, carry):
                base = sid * edges_per_tile + wi * w
                pltpu.sync_copy(dst_hbm.at[pl.ds(base, w)], idx_v)
                pltpu.sync_copy(upd_hbm.at[pl.ds(base, w)], buf_v)
                pltpu.sync_copy(buf_v, acc.at[idx_v], add=True)
                return carry
            lax.fori_loop(0, nwin, body, 0)

        @pl.when(cid == 0)
        def _():
            run(upd_a)

        @pl.when(cid == 1)
        def _():
            run(upd_b)

        plsc.subcore_barrier()

        @pl.when(cid == 0)
        def _():
            rows_copy(acc, out_a)

        @pl.when(cid == 1)
        def _():
            rows_copy(acc, out_b)

    return k


def _make_sc_gather_pre(e, h, w):
    mesh = plsc.VectorSubcoreMesh(core_axis_name="c", subcore_axis_name="s")
    nworkers = 32
    epw = e // nworkers
    nwin = epw // w

    @functools.partial(
        pl.kernel,
        mesh=mesh,
        out_type=jax.ShapeDtypeStruct((e, h), jnp.float32),
        scratch_types=[
            pltpu.VMEM((w,), jnp.int32),
            pltpu.VMEM((w,), jnp.int32),
            pltpu.VMEM((w, h), jnp.float32),
            pltpu.VMEM((w, h), jnp.float32),
            pltpu.SemaphoreType.DMA,
            pltpu.SemaphoreType.DMA,
        ],
    )
    def k(a_hbm, b_hbm, dst_hbm, src_hbm, out_hbm,
          idx_d, idx_s, buf_a, buf_b, sem_a, sem_b):
        cid = lax.axis_index("c")
        sid = lax.axis_index("s")
        wid = sid * 2 + cid

        def body(wi, carry):
            base = wid * epw + wi * w
            pltpu.sync_copy(dst_hbm.at[pl.ds(base, w)], idx_d)
            pltpu.sync_copy(src_hbm.at[pl.ds(base, w)], idx_s)
            cp_a = pltpu.async_copy(a_hbm.at[idx_d], buf_a, sem_a)
            cp_b = pltpu.async_copy(b_hbm.at[idx_s], buf_b, sem_b)
            cp_a.wait()
            cp_b.wait()

            def add_row(r, c2):
                for c8 in range(h // 16):
                    sl = pl.ds(c8 * 16, 16)
                    buf_a[r, sl] = buf_a[r, sl] + buf_b[r, sl]
                return c2

            lax.fori_loop(0, w, add_row, 0)
            pltpu.sync_copy(buf_a, out_hbm.at[pl.ds(base, w)])
            return carry

        lax.fori_loop(0, nwin, body, 0)

    return k


def _make_sc_gather_vj(e, h, w):
    mesh = plsc.VectorSubcoreMesh(core_axis_name="c", subcore_axis_name="s")
    nworkers = 32
    epw = e // nworkers
    nwin = epw // w

    @functools.partial(
        pl.kernel,
        mesh=mesh,
        out_type=[jax.ShapeDtypeStruct((e, h), jnp.float32)] * 3,
        scratch_types=[
            pltpu.VMEM((w,), jnp.int32),
            pltpu.VMEM((w, h), jnp.float32),
            pltpu.SemaphoreType.DMA,
        ],
    )
    def k(v0_hbm, v1_hbm, v2_hbm, src_hbm, o0, o1, o2, idx_s, buf, sem):
        cid = lax.axis_index("c")
        sid = lax.axis_index("s")
        wid = sid * 2 + cid

        def run(tab, out):
            def body(wi, carry):
                base = wid * epw + wi * w
                pltpu.sync_copy(src_hbm.at[pl.ds(base, w)], idx_s)
                pltpu.async_copy(tab.at[idx_s], buf, sem).wait()
                pltpu.sync_copy(buf, out.at[pl.ds(base, w)])
                return carry

            lax.fori_loop(0, nwin, body, 0)

        run(v0_hbm, o0)
        run(v1_hbm, o1)
        run(v2_hbm, o2)

    return k


def kernel(s, v, edge_attr, edge_vec,
           s_proj_w1, s_proj_b1, s_proj_w2, s_proj_b2,
           edge_proj_w1, edge_proj_b1, edge_proj_w2, edge_proj_b2,
           final_s_w, final_s_b, final_v_w,
           U_w, V_w,
           s_mlp_w1, s_mlp_b1, s_mlp_w2, s_mlp_b2,
           edge_index):
    n, h = s.shape
    e = edge_attr.shape[0]
    src = edge_index[0]
    dst = edge_index[1]

    w1a_t = s_proj_w1[:, :h].T
    w1b_t = s_proj_w1[:, h:].T
    a_tab, b_tab = _node_pre(s, w1a_t, w1b_t, bn=1000)

    pre = _make_sc_gather_pre(e, h, w=200)(a_tab, b_tab, dst, src)
    v_t = jnp.transpose(v, (2, 0, 1))
    vj0, vj1, vj2 = _make_sc_gather_vj(e, h, w=1000)(v_t[0], v_t[1], v_t[2], src)

    ds_msg, dvm0, dvm1, dvm2 = _edge_stage(
        pre, edge_attr, vj0, vj1, vj2, edge_vec,
        s_proj_b1.reshape(1, h), s_proj_w2.T, s_proj_b2.reshape(1, 3 * h),
        edge_proj_w1.T, edge_proj_b1.reshape(1, h),
        edge_proj_w2.T, edge_proj_b2.reshape(1, 3 * h), be=1000)

    scatter_pair = _make_sc_scatter_pair(e, n, h, w=200)
    zeros_nh = jnp.zeros((n, h), jnp.float32)
    ds, dv0 = scatter_pair(ds_msg, dvm0, dst, zeros_nh)
    dv1, dv2 = scatter_pair(dvm1, dvm2, dst, zeros_nh)

    m1a_t = s_mlp_w1[:, :h].T
    m1b_t = s_mlp_w1[:, h:].T
    s_out, vo0, vo1, vo2 = _node_update(
        s, v_t[0], v_t[1], v_t[2], ds, dv0, dv1, dv2,
        final_s_w.T, final_s_b.reshape(1, h), final_v_w.T,
        U_w.T, V_w.T, m1a_t, m1b_t, s_mlp_b1.reshape(1, h),
        s_mlp_w2.T, s_mlp_b2.reshape(1, 3 * h), bn=1000)

    v_out = jnp.stack([vo0, vo1, vo2], axis=-1)
    return (s_out, v_out)


# speedup vs baseline: 21.9113x; 1.2430x over previous
import functools

import jax
import jax.numpy as jnp
from jax import lax
from jax.experimental import pallas as pl
from jax.experimental.pallas import tpu as pltpu
from jax.experimental.pallas import tpu_sc as plsc

RES = 0.1
CLAMP = 100.0


def _silu(x):
    return x * jax.nn.sigmoid(x)


def _node_pre_body(s_ref, w1a_ref, w1b_ref, a_ref, b_ref):
    s = s_ref[...]
    a_ref[...] = jnp.dot(s, w1a_ref[...], preferred_element_type=jnp.float32)
    b_ref[...] = jnp.dot(s, w1b_ref[...], preferred_element_type=jnp.float32)


def _node_pre(s, w1a_t, w1b_t, bn):
    n, h = s.shape
    grid = (n // bn,)
    return pl.pallas_call(
        _node_pre_body,
        grid=grid,
        in_specs=[
            pl.BlockSpec((bn, h), lambda i: (i, 0)),
            pl.BlockSpec((h, h), lambda i: (0, 0)),
            pl.BlockSpec((h, h), lambda i: (0, 0)),
        ],
        out_specs=[
            pl.BlockSpec((bn, h), lambda i: (i, 0)),
            pl.BlockSpec((bn, h), lambda i: (i, 0)),
        ],
        out_shape=[
            jax.ShapeDtypeStruct((n, h), jnp.float32),
            jax.ShapeDtypeStruct((n, h), jnp.float32),
        ],
    )(s, w1a_t, w1b_t)


def _edge_body(pre_ref, ea_ref, vj0_ref, vj1_ref, vj2_ref, ev_ref,
               b1_ref, w2t_ref, b2_ref,
               ew1t_ref, eb1_ref, ew2t_ref, eb2_ref,
               ds_ref, dv0_ref, dv1_ref, dv2_ref):
    h = pre_ref.shape[1]
    h1 = _silu(pre_ref[...] + b1_ref[...])
    phis = jnp.dot(h1, w2t_ref[...], preferred_element_type=jnp.float32) + b2_ref[...]
    he = _silu(jnp.dot(ea_ref[...], ew1t_ref[...],
                       preferred_element_type=jnp.float32) + eb1_ref[...])
    phie = jnp.dot(he, ew2t_ref[...], preferred_element_type=jnp.float32) + eb2_ref[...]
    phi = phis * phie
    phi1 = phi[:, :h]
    phi2 = phi[:, h:2 * h]
    phi3 = phi[:, 2 * h:]
    ev = ev_ref[...]
    ds_ref[...] = phi1
    dv0_ref[...] = phi2 * vj0_ref[...] + phi3 * ev[:, 0:1]
    dv1_ref[...] = phi2 * vj1_ref[...] + phi3 * ev[:, 1:2]
    dv2_ref[...] = phi2 * vj2_ref[...] + phi3 * ev[:, 2:3]


def _edge_stage(pre, ea, vj0, vj1, vj2, ev,
                b1, w2t, b2, ew1t, eb1, ew2t, eb2, be):
    e, h = pre.shape
    ed = ea.shape[1]
    grid = (e // be,)
    edge_block = lambda i: (i, 0)
    full = lambda i: (0, 0)
    return pl.pallas_call(
        _edge_body,
        grid=grid,
        in_specs=[
            pl.BlockSpec((be, h), edge_block),
            pl.BlockSpec((be, ed), edge_block),
            pl.BlockSpec((be, h), edge_block),
            pl.BlockSpec((be, h), edge_block),
            pl.BlockSpec((be, h), edge_block),
            pl.BlockSpec((be, 3), edge_block),
            pl.BlockSpec((1, h), full),
            pl.BlockSpec((h, 3 * h), full),
            pl.BlockSpec((1, 3 * h), full),
            pl.BlockSpec((ed, h), full),
            pl.BlockSpec((1, h), full),
            pl.BlockSpec((h, 3 * h), full),
            pl.BlockSpec((1, 3 * h), full),
        ],
        out_specs=[pl.BlockSpec((be, h), edge_block)] * 4,
        out_shape=[jax.ShapeDtypeStruct((e, h), jnp.float32)] * 4,
    )(pre, ea, vj0, vj1, vj2, ev, b1, w2t, b2, ew1t, eb1, ew2t, eb2)


def _node_update_body(s_ref, v0_ref, v1_ref, v2_ref,
                      ds_ref, dv0_ref, dv1_ref, dv2_ref,
                      fst_ref, fsb_ref, fvt_ref,
                      ut_ref, vt_ref,
                      m1at_ref, m1bt_ref, m1b_ref, m2t_ref, m2b_ref,
                      so_ref, vo0_ref, vo1_ref, vo2_ref):
    h = s_ref.shape[1]
    ds = jnp.dot(ds_ref[...], fst_ref[...], preferred_element_type=jnp.float32) + fsb_ref[...]
    s1 = s_ref[...] + RES * ds
    fvt = fvt_ref[...]
    v0 = v0_ref[...] + RES * jnp.dot(dv0_ref[...], fvt, preferred_element_type=jnp.float32)
    v1 = v1_ref[...] + RES * jnp.dot(dv1_ref[...], fvt, preferred_element_type=jnp.float32)
    v2 = v2_ref[...] + RES * jnp.dot(dv2_ref[...], fvt, preferred_element_type=jnp.float32)
    ut = ut_ref[...]
    vt = vt_ref[...]
    uv0 = jnp.dot(v0, ut, preferred_element_type=jnp.float32)
    uv1 = jnp.dot(v1, ut, preferred_element_type=jnp.float32)
    uv2 = jnp.dot(v2, ut, preferred_element_type=jnp.float32)
    vv0 = jnp.dot(v0, vt, preferred_element_type=jnp.float32)
    vv1 = jnp.dot(v1, vt, preferred_element_type=jnp.float32)
    vv2 = jnp.dot(v2, vt, preferred_element_type=jnp.float32)
    vnorm = jnp.sqrt(vv0 * vv0 + vv1 * vv1 + vv2 * vv2)
    m1 = (jnp.dot(s1, m1at_ref[...], preferred_element_type=jnp.float32)
          + jnp.dot(vnorm, m1bt_ref[...], preferred_element_type=jnp.float32)
          + m1b_ref[...])
    a = jnp.dot(_silu(m1), m2t_ref[...], preferred_element_type=jnp.float32) + m2b_ref[...]
    a1 = a[:, :h]
    a2 = a[:, h:2 * h]
    a3 = a[:, 2 * h:]
    so_ref[...] = jnp.clip(s1 + RES * (a1 + a2 * vnorm), -CLAMP, CLAMP)
    vo0_ref[...] = jnp.clip(v0 + RES * (a3 * uv0), -CLAMP, CLAMP)
    vo1_ref[...] = jnp.clip(v1 + RES * (a3 * uv1), -CLAMP, CLAMP)
    vo2_ref[...] = jnp.clip(v2 + RES * (a3 * uv2), -CLAMP, CLAMP)


def _node_update(s, v0, v1, v2, ds, dv0, dv1, dv2,
                 fst, fsb, fvt, ut, vt, m1at, m1bt, m1b, m2t, m2b, bn):
    n, h = s.shape
    grid = (n // bn,)
    nb = lambda i: (i, 0)
    full = lambda i: (0, 0)
    return pl.pallas_call(
        _node_update_body,
        grid=grid,
        in_specs=[pl.BlockSpec((bn, h), nb)] * 8 + [
            pl.BlockSpec((h, h), full),
            pl.BlockSpec((1, h), full),
            pl.BlockSpec((h, h), full),
            pl.BlockSpec((h, h), full),
            pl.BlockSpec((h, h), full),
            pl.BlockSpec((h, h), full),
            pl.BlockSpec((h, h), full),
            pl.BlockSpec((1, h), full),
            pl.BlockSpec((h, 3 * h), full),
            pl.BlockSpec((1, 3 * h), full),
        ],
        out_specs=[pl.BlockSpec((bn, h), nb)] * 4,
        out_shape=[jax.ShapeDtypeStruct((n, h), jnp.float32)] * 4,
    )(s, v0, v1, v2, ds, dv0, dv1, dv2,
      fst, fsb, fvt, ut, vt, m1at, m1bt, m1b, m2t, m2b)


def _make_sc_scatter_pair(e, n, h, w):
    mesh = plsc.VectorSubcoreMesh(core_axis_name="c", subcore_axis_name="s")
    nsub = 16
    rpt = (n // nsub) // 8 * 8
    tail = n - nsub * rpt
    edges_per_tile = e // nsub
    nwin = edges_per_tile // w

    @functools.partial(
        pl.kernel,
        mesh=mesh,
        out_type=[jax.ShapeDtypeStruct((n, h), jnp.float32)] * 2,
        scratch_types=[
            pltpu.VMEM_SHARED((n, h), jnp.float32),
            pltpu.VMEM((w,), jnp.int32),
            pltpu.VMEM((w,), jnp.int32),
            pltpu.VMEM((w, h), jnp.float32),
            pltpu.VMEM((w, h), jnp.float32),
            pltpu.SemaphoreType.DMA,
            pltpu.SemaphoreType.DMA,
            pltpu.SemaphoreType.DMA,
            pltpu.SemaphoreType.DMA,
        ],
    )
    def k(upd_a, upd_b, dst_hbm, zeros_hbm, out_a, out_b, acc,
          idx0, idx1, buf0, buf1, si0, si1, su0, su1):
        cid = lax.axis_index("c")
        sid = lax.axis_index("s")
        r0 = sid * rpt

        def rows_copy(src, dst_ref):
            pltpu.sync_copy(src.at[pl.ds(r0, rpt)], dst_ref.at[pl.ds(r0, rpt)])
            if tail:
                @pl.when(sid == 0)
                def _():
                    pltpu.sync_copy(src.at[pl.ds(nsub * rpt, tail)],
                                    dst_ref.at[pl.ds(nsub * rpt, tail)])

        rows_copy(zeros_hbm, acc)
        plsc.subcore_barrier()

        def run(upd_hbm):
            slots = ((idx0, buf0, si0, su0), (idx1, buf1, si1, su1))

            def start(wi, slot):
                idx_v, buf_v, sem_i, sem_u = slot
                base = sid * edges_per_tile + wi * w
                pltpu.async_copy(dst_hbm.at[pl.ds(base, w)], idx_v, sem_i)
                pltpu.async_copy(upd_hbm.at[pl.ds(base, w)], buf_v, sem_u)

            def finish(slot):
                idx_v, buf_v, sem_i, sem_u = slot
                pltpu.make_async_copy(dst_hbm.at[pl.ds(0, w)], idx_v, sem_i).wait()
                pltpu.make_async_copy(upd_hbm.at[pl.ds(0, w)], buf_v, sem_u).wait()
                pltpu.sync_copy(buf_v, acc.at[idx_v], add=True)

            start(0, slots[0])

            def body(i, carry):
                w2 = 2 * i + 2
                start(2 * i + 1, slots[1])
                finish(slots[0])

                @pl.when(w2 < nwin)
                def _():
                    start(w2, slots[0])

                finish(slots[1])
                return carry

            lax.fori_loop(0, nwin // 2, body, 0)
            if nwin % 2:
                finish(slots[0])

        @pl.when(cid == 0)
        def _():
            run(upd_a)

        @pl.when(cid == 1)
        def _():
            run(upd_b)

        plsc.subcore_barrier()

        @pl.when(cid == 0)
        def _():
            rows_copy(acc, out_a)

        @pl.when(cid == 1)
        def _():
            rows_copy(acc, out_b)

    return k


def _make_sc_gather_pre(e, h, w):
    mesh = plsc.VectorSubcoreMesh(core_axis_name="c", subcore_axis_name="s")
    nworkers = 32
    epw = e // nworkers
    nwin = epw // w

    @functools.partial(
        pl.kernel,
        mesh=mesh,
        out_type=jax.ShapeDtypeStruct((e, h), jnp.float32),
        scratch_types=[
            pltpu.VMEM((w,), jnp.int32),
            pltpu.VMEM((w,), jnp.int32),
            pltpu.VMEM((w, h), jnp.float32),
            pltpu.VMEM((w, h), jnp.float32),
            pltpu.VMEM((w,), jnp.int32),
            pltpu.VMEM((w,), jnp.int32),
            pltpu.VMEM((w, h), jnp.float32),
            pltpu.VMEM((w, h), jnp.float32),
            pltpu.SemaphoreType.DMA,
            pltpu.SemaphoreType.DMA,
            pltpu.SemaphoreType.DMA,
            pltpu.SemaphoreType.DMA,
        ],
    )
    def k(a_hbm, b_hbm, dst_hbm, src_hbm, out_hbm,
          idd0, ids0, bufa0, bufb0, idd1, ids1, bufa1, bufb1,
          sa0, sb0, sa1, sb1):
        cid = lax.axis_index("c")
        sid = lax.axis_index("s")
        wid = sid * 2 + cid
        slots = ((idd0, ids0, bufa0, bufb0, sa0, sb0),
                 (idd1, ids1, bufa1, bufb1, sa1, sb1))

        def start(wi, slot):
            idx_d, idx_s, buf_a, buf_b, sem_a, sem_b = slot
            base = wid * epw + wi * w
            pltpu.sync_copy(dst_hbm.at[pl.ds(base, w)], idx_d)
            pltpu.sync_copy(src_hbm.at[pl.ds(base, w)], idx_s)
            pltpu.async_copy(a_hbm.at[idx_d], buf_a, sem_a)
            pltpu.async_copy(b_hbm.at[idx_s], buf_b, sem_b)

        def finish(wi, slot):
            idx_d, idx_s, buf_a, buf_b, sem_a, sem_b = slot
            base = wid * epw + wi * w
            pltpu.make_async_copy(a_hbm.at[idx_d], buf_a, sem_a).wait()
            pltpu.make_async_copy(b_hbm.at[idx_s], buf_b, sem_b).wait()

            def add_row(r, c2):
                for c8 in range(h // 16):
                    sl = pl.ds(c8 * 16, 16)
                    buf_a[r, sl] = buf_a[r, sl] + buf_b[r, sl]
                return c2

            lax.fori_loop(0, w, add_row, 0)
            pltpu.sync_copy(buf_a, out_hbm.at[pl.ds(base, w)])

        start(0, slots[0])

        def body(i, carry):
            w2 = 2 * i + 2
            start(2 * i + 1, slots[1])
            finish(2 * i, slots[0])

            @pl.when(w2 < nwin)
            def _():
                start(w2, slots[0])

            finish(2 * i + 1, slots[1])
            return carry

        lax.fori_loop(0, nwin // 2, body, 0)

    return k


def _make_sc_gather_vj(e, h, w):
    mesh = plsc.VectorSubcoreMesh(core_axis_name="c", subcore_axis_name="s")
    nworkers = 32
    epw = e // nworkers
    nwin = epw // w

    @functools.partial(
        pl.kernel,
        mesh=mesh,
        out_type=[jax.ShapeDtypeStruct((e, h), jnp.float32)] * 3,
        scratch_types=[
            pltpu.VMEM((w,), jnp.int32),
            pltpu.VMEM((w, h), jnp.float32),
            pltpu.VMEM((w,), jnp.int32),
            pltpu.VMEM((w, h), jnp.float32),
            pltpu.SemaphoreType.DMA,
            pltpu.SemaphoreType.DMA,
        ],
    )
    def k(v0_hbm, v1_hbm, v2_hbm, src_hbm, o0, o1, o2,
          idx0, buf0, idx1, buf1, sem0, sem1):
        cid = lax.axis_index("c")
        sid = lax.axis_index("s")
        wid = sid * 2 + cid
        slots = ((idx0, buf0, sem0), (idx1, buf1, sem1))

        def run(tab, out):
            def start(wi, slot):
                idx_s, buf, sem = slot
                base = wid * epw + wi * w
                pltpu.sync_copy(src_hbm.at[pl.ds(base, w)], idx_s)
                pltpu.async_copy(tab.at[idx_s], buf, sem)

            def finish(wi, slot):
                idx_s, buf, sem = slot
                base = wid * epw + wi * w
                pltpu.make_async_copy(tab.at[idx_s], buf, sem).wait()
                pltpu.sync_copy(buf, out.at[pl.ds(base, w)])

            start(0, slots[0])

            def body(i, carry):
                start(2 * i + 1, slots[1])
                finish(2 * i, slots[0])
                start(2 * i + 2, slots[0])
                finish(2 * i + 1, slots[1])
                return carry

            lax.fori_loop(0, nwin // 2, body, 0)
            finish(nwin - 1, slots[0])

        run(v0_hbm, o0)
        run(v1_hbm, o1)
        run(v2_hbm, o2)

    return k


def kernel(s, v, edge_attr, edge_vec,
           s_proj_w1, s_proj_b1, s_proj_w2, s_proj_b2,
           edge_proj_w1, edge_proj_b1, edge_proj_w2, edge_proj_b2,
           final_s_w, final_s_b, final_v_w,
           U_w, V_w,
           s_mlp_w1, s_mlp_b1, s_mlp_w2, s_mlp_b2,
           edge_index):
    n, h = s.shape
    e = edge_attr.shape[0]
    src = edge_index[0]
    dst = edge_index[1]

    w1a_t = s_proj_w1[:, :h].T
    w1b_t = s_proj_w1[:, h:].T
    a_tab, b_tab = _node_pre(s, w1a_t, w1b_t, bn=1000)

    pre = _make_sc_gather_pre(e, h, w=200)(a_tab, b_tab, dst, src)
    v_t = jnp.transpose(v, (2, 0, 1))
    vj0, vj1, vj2 = _make_sc_gather_vj(e, h, w=400)(v_t[0], v_t[1], v_t[2], src)

    ds_msg, dvm0, dvm1, dvm2 = _edge_stage(
        pre, edge_attr, vj0, vj1, vj2, edge_vec,
        s_proj_b1.reshape(1, h), s_proj_w2.T, s_proj_b2.reshape(1, 3 * h),
        edge_proj_w1.T, edge_proj_b1.reshape(1, h),
        edge_proj_w2.T, edge_proj_b2.reshape(1, 3 * h), be=1000)

    scatter_pair = _make_sc_scatter_pair(e, n, h, w=160)
    zeros_nh = jnp.zeros((n, h), jnp.float32)
    ds, dv0 = scatter_pair(ds_msg, dvm0, dst, zeros_nh)
    dv1, dv2 = scatter_pair(dvm1, dvm2, dst, zeros_nh)

    m1a_t = s_mlp_w1[:, :h].T
    m1b_t = s_mlp_w1[:, h:].T
    s_out, vo0, vo1, vo2 = _node_update(
        s, v_t[0], v_t[1], v_t[2], ds, dv0, dv1, dv2,
        final_s_w.T, final_s_b.reshape(1, h), final_v_w.T,
        U_w.T, V_w.T, m1a_t, m1b_t, s_mlp_b1.reshape(1, h),
        s_mlp_w2.T, s_mlp_b2.reshape(1, 3 * h), bn=1000)

    v_out = jnp.stack([vo0, vo1, vo2], axis=-1)
    return (s_out, v_out)


# speedup vs baseline: 25.0966x; 1.1454x over previous
import functools

import jax
import jax.numpy as jnp
from jax import lax
from jax.experimental import pallas as pl
from jax.experimental.pallas import tpu as pltpu
from jax.experimental.pallas import tpu_sc as plsc

RES = 0.1
CLAMP = 100.0


def _silu(x):
    return x * jax.nn.sigmoid(x)


def _pack2(lo, hi):
    ul = jax.lax.bitcast_convert_type(
        lo.astype(jnp.bfloat16), jnp.uint16).astype(jnp.uint32)
    uh = jax.lax.bitcast_convert_type(
        hi.astype(jnp.bfloat16), jnp.uint16).astype(jnp.uint32)
    return jax.lax.bitcast_convert_type((uh << 16) | ul, jnp.float32)


def _unpack2(x):
    u = jax.lax.bitcast_convert_type(x, jnp.uint32)
    lo = jax.lax.bitcast_convert_type(u << 16, jnp.float32)
    hi = jax.lax.bitcast_convert_type(u & jnp.uint32(0xFFFF0000), jnp.float32)
    return lo, hi


def _node_pre_body(s_ref, w1a_ref, w1b_ref, v0_ref, v1_ref, v2_ref,
                   a_ref, tb2_ref, t01_ref):
    s = s_ref[...]
    a_ref[...] = jnp.dot(s, w1a_ref[...], preferred_element_type=jnp.float32)
    b = jnp.dot(s, w1b_ref[...], preferred_element_type=jnp.float32)
    tb2_ref[...] = _pack2(b, v2_ref[...])
    t01_ref[...] = _pack2(v0_ref[...], v1_ref[...])


def _node_pre(s, w1a_t, w1b_t, v0, v1, v2, bn):
    n, h = s.shape
    grid = (n // bn,)
    nb = lambda i: (i, 0)
    return pl.pallas_call(
        _node_pre_body,
        grid=grid,
        in_specs=[
            pl.BlockSpec((bn, h), nb),
            pl.BlockSpec((h, h), lambda i: (0, 0)),
            pl.BlockSpec((h, h), lambda i: (0, 0)),
            pl.BlockSpec((bn, h), nb),
            pl.BlockSpec((bn, h), nb),
            pl.BlockSpec((bn, h), nb),
        ],
        out_specs=[pl.BlockSpec((bn, h), nb)] * 3,
        out_shape=[jax.ShapeDtypeStruct((n, h), jnp.float32)] * 3,
    )(s, w1a_t, w1b_t, v0, v1, v2)


def _edge_body(prea_ref, tb2_ref, t01_ref, ea_ref, ev_ref,
               b1_ref, w2t_ref, b2_ref,
               ew1t_ref, eb1_ref, ew2t_ref, eb2_ref,
               ds_ref, dv0_ref, dv1_ref, dv2_ref):
    h = prea_ref.shape[1]
    pre_b, vj2 = _unpack2(tb2_ref[...])
    vj0, vj1 = _unpack2(t01_ref[...])
    h1 = _silu(prea_ref[...] + pre_b + b1_ref[...])
    phis = jnp.dot(h1.astype(jnp.bfloat16), w2t_ref[...],
                   preferred_element_type=jnp.float32) + b2_ref[...]
    he = _silu(jnp.dot(ea_ref[...], ew1t_ref[...],
                       preferred_element_type=jnp.float32) + eb1_ref[...])
    phie = jnp.dot(he.astype(jnp.bfloat16), ew2t_ref[...],
                   preferred_element_type=jnp.float32) + eb2_ref[...]
    phi = phis * phie
    ev = ev_ref[...]
    ds_ref[...] = phi[:, :h]
    phi2 = phi[:, h:2 * h]
    phi3 = phi[:, 2 * h:]
    dv0_ref[...] = phi2 * vj0 + phi3 * ev[:, 0:1]
    dv1_ref[...] = phi2 * vj1 + phi3 * ev[:, 1:2]
    dv2_ref[...] = phi2 * vj2 + phi3 * ev[:, 2:3]


def _edge_stage(prea, tb2, t01, ea, ev,
                b1, w2t, b2, ew1t, eb1, ew2t, eb2, be):
    e, h = prea.shape
    ed = ea.shape[1]
    grid = (e // be,)
    edge_block = lambda i: (i, 0)
    full = lambda i: (0, 0)
    return pl.pallas_call(
        _edge_body,
        grid=grid,
        in_specs=[
            pl.BlockSpec((be, h), edge_block),
            pl.BlockSpec((be, h), edge_block),
            pl.BlockSpec((be, h), edge_block),
            pl.BlockSpec((be, ed), edge_block),
            pl.BlockSpec((be, 3), edge_block),
            pl.BlockSpec((1, h), full),
            pl.BlockSpec((h, 3 * h), full),
            pl.BlockSpec((1, 3 * h), full),
            pl.BlockSpec((ed, h), full),
            pl.BlockSpec((1, h), full),
            pl.BlockSpec((h, 3 * h), full),
            pl.BlockSpec((1, 3 * h), full),
        ],
        out_specs=[pl.BlockSpec((be, h), edge_block)] * 4,
        out_shape=[jax.ShapeDtypeStruct((e, h), jnp.float32)] * 4,
    )(prea, tb2, t01, ea, ev, b1, w2t, b2, ew1t, eb1, ew2t, eb2)


def _node_update_body(s_ref, v0_ref, v1_ref, v2_ref,
                      ds_ref, dv0_ref, dv1_ref, dv2_ref,
                      fst_ref, fsb_ref, fvt_ref,
                      ut_ref, vt_ref,
                      m1at_ref, m1bt_ref, m1b_ref, m2t_ref, m2b_ref,
                      so_ref, vo0_ref, vo1_ref, vo2_ref):
    h = s_ref.shape[1]
    ds = jnp.dot(ds_ref[...], fst_ref[...], preferred_element_type=jnp.float32) + fsb_ref[...]
    s1 = s_ref[...] + RES * ds
    fvt = fvt_ref[...]
    v0 = v0_ref[...] + RES * jnp.dot(dv0_ref[...], fvt, preferred_element_type=jnp.float32)
    v1 = v1_ref[...] + RES * jnp.dot(dv1_ref[...], fvt, preferred_element_type=jnp.float32)
    v2 = v2_ref[...] + RES * jnp.dot(dv2_ref[...], fvt, preferred_element_type=jnp.float32)
    ut = ut_ref[...]
    vt = vt_ref[...]
    uv0 = jnp.dot(v0, ut, preferred_element_type=jnp.float32)
    uv1 = jnp.dot(v1, ut, preferred_element_type=jnp.float32)
    uv2 = jnp.dot(v2, ut, preferred_element_type=jnp.float32)
    vv0 = jnp.dot(v0, vt, preferred_element_type=jnp.float32)
    vv1 = jnp.dot(v1, vt, preferred_element_type=jnp.float32)
    vv2 = jnp.dot(v2, vt, preferred_element_type=jnp.float32)
    vnorm = jnp.sqrt(vv0 * vv0 + vv1 * vv1 + vv2 * vv2)
    m1 = (jnp.dot(s1, m1at_ref[...], preferred_element_type=jnp.float32)
          + jnp.dot(vnorm, m1bt_ref[...], preferred_element_type=jnp.float32)
          + m1b_ref[...])
    a = jnp.dot(_silu(m1), m2t_ref[...], preferred_element_type=jnp.float32) + m2b_ref[...]
    a1 = a[:, :h]
    a2 = a[:, h:2 * h]
    a3 = a[:, 2 * h:]
    so_ref[...] = jnp.clip(s1 + RES * (a1 + a2 * vnorm), -CLAMP, CLAMP)
    vo0_ref[...] = jnp.clip(v0 + RES * (a3 * uv0), -CLAMP, CLAMP)
    vo1_ref[...] = jnp.clip(v1 + RES * (a3 * uv1), -CLAMP, CLAMP)
    vo2_ref[...] = jnp.clip(v2 + RES * (a3 * uv2), -CLAMP, CLAMP)


def _node_update(s, v0, v1, v2, ds, dv0, dv1, dv2,
                 fst, fsb, fvt, ut, vt, m1at, m1bt, m1b, m2t, m2b, bn):
    n, h = s.shape
    grid = (n // bn,)
    nb = lambda i: (i, 0)
    full = lambda i: (0, 0)
    return pl.pallas_call(
        _node_update_body,
        grid=grid,
        in_specs=[pl.BlockSpec((bn, h), nb)] * 8 + [
            pl.BlockSpec((h, h), full),
            pl.BlockSpec((1, h), full),
            pl.BlockSpec((h, h), full),
            pl.BlockSpec((h, h), full),
            pl.BlockSpec((h, h), full),
            pl.BlockSpec((h, h), full),
            pl.BlockSpec((h, h), full),
            pl.BlockSpec((1, h), full),
            pl.BlockSpec((h, 3 * h), full),
            pl.BlockSpec((1, 3 * h), full),
        ],
        out_specs=[pl.BlockSpec((bn, h), nb)] * 4,
        out_shape=[jax.ShapeDtypeStruct((n, h), jnp.float32)] * 4,
    )(s, v0, v1, v2, ds, dv0, dv1, dv2,
      fst, fsb, fvt, ut, vt, m1at, m1bt, m1b, m2t, m2b)


def _make_sc_scatter_pair(e, n, h, w):
    mesh = plsc.VectorSubcoreMesh(core_axis_name="c", subcore_axis_name="s")
    nsub = 16
    rpt = (n // nsub) // 8 * 8
    tail = n - nsub * rpt
    edges_per_tile = e // nsub
    nwin = edges_per_tile // w

    @functools.partial(
        pl.kernel,
        mesh=mesh,
        out_type=[jax.ShapeDtypeStruct((n, h), jnp.float32)] * 2,
        scratch_types=[
            pltpu.VMEM_SHARED((n, h), jnp.float32),
            pltpu.VMEM((w,), jnp.int32),
            pltpu.VMEM((w,), jnp.int32),
            pltpu.VMEM((w, h), jnp.float32),
            pltpu.VMEM((w, h), jnp.float32),
            pltpu.SemaphoreType.DMA,
            pltpu.SemaphoreType.DMA,
            pltpu.SemaphoreType.DMA,
            pltpu.SemaphoreType.DMA,
        ],
    )
    def k(upd_a, upd_b, dst_hbm, zeros_hbm, out_a, out_b, acc,
          idx0, idx1, buf0, buf1, si0, si1, su0, su1):
        cid = lax.axis_index("c")
        sid = lax.axis_index("s")
        r0 = sid * rpt

        def rows_copy(src, dst_ref):
            pltpu.sync_copy(src.at[pl.ds(r0, rpt)], dst_ref.at[pl.ds(r0, rpt)])
            if tail:
                @pl.when(sid == 0)
                def _():
                    pltpu.sync_copy(src.at[pl.ds(nsub * rpt, tail)],
                                    dst_ref.at[pl.ds(nsub * rpt, tail)])

        rows_copy(zeros_hbm, acc)
        plsc.subcore_barrier()

        def run(upd_hbm):
            slots = ((idx0, buf0, si0, su0), (idx1, buf1, si1, su1))

            def start(wi, slot):
                idx_v, buf_v, sem_i, sem_u = slot
                base = sid * edges_per_tile + wi * w
                pltpu.async_copy(dst_hbm.at[pl.ds(base, w)], idx_v, sem_i)
                pltpu.async_copy(upd_hbm.at[pl.ds(base, w)], buf_v, sem_u)

            def finish(slot):
                idx_v, buf_v, sem_i, sem_u = slot
                pltpu.make_async_copy(dst_hbm.at[pl.ds(0, w)], idx_v, sem_i).wait()
                pltpu.make_async_copy(upd_hbm.at[pl.ds(0, w)], buf_v, sem_u).wait()
                pltpu.sync_copy(buf_v, acc.at[idx_v], add=True)

            start(0, slots[0])

            def body(i, carry):
                w2 = 2 * i + 2
                start(2 * i + 1, slots[1])
                finish(slots[0])

                @pl.when(w2 < nwin)
                def _():
                    start(w2, slots[0])

                finish(slots[1])
                return carry

            lax.fori_loop(0, nwin // 2, body, 0)
            if nwin % 2:
                finish(slots[0])

        @pl.when(cid == 0)
        def _():
            run(upd_a)

        @pl.when(cid == 1)
        def _():
            run(upd_b)

        plsc.subcore_barrier()

        @pl.when(cid == 0)
        def _():
            rows_copy(acc, out_a)

        @pl.when(cid == 1)
        def _():
            rows_copy(acc, out_b)

    return k


def _make_sc_gather_tabs(e, h, w, sels, dtype=jnp.float32):
    mesh = plsc.VectorSubcoreMesh(core_axis_name="c", subcore_axis_name="s")
    nworkers = 32
    epw = e // nworkers
    nwin = epw // w
    nt = len(sels)

    @functools.partial(
        pl.kernel,
        mesh=mesh,
        out_type=[jax.ShapeDtypeStruct((e, h), dtype)] * nt,
        scratch_types=[
            pltpu.VMEM((w,), jnp.int32),
            pltpu.VMEM((w, h), dtype),
            pltpu.VMEM((w,), jnp.int32),
            pltpu.VMEM((w, h), dtype),
            pltpu.SemaphoreType.DMA,
            pltpu.SemaphoreType.DMA,
        ],
    )
    def k(*refs):
        tabs = refs[:nt]
        dst_hbm, src_hbm = refs[nt], refs[nt + 1]
        outs = refs[nt + 2:nt + 2 + nt]
        idx0, buf0, idx1, buf1, sem0, sem1 = refs[nt + 2 + nt:]
        cid = lax.axis_index("c")
        sid = lax.axis_index("s")
        wid = sid * 2 + cid
        slots = ((idx0, buf0, sem0), (idx1, buf1, sem1))

        def run(tab, out, idx_hbm):
            def start(wi, slot):
                idx_v, buf, sem = slot
                base = wid * epw + wi * w
                pltpu.sync_copy(idx_hbm.at[pl.ds(base, w)], idx_v)
                pltpu.async_copy(tab.at[idx_v], buf, sem)

            def finish(wi, slot):
                idx_v, buf, sem = slot
                base = wid * epw + wi * w
                pltpu.make_async_copy(tab.at[idx_v], buf, sem).wait()
                pltpu.sync_copy(buf, out.at[pl.ds(base, w)])

            start(0, slots[0])

            def body(i, carry):
                w2 = 2 * i + 2
                start(2 * i + 1, slots[1])
                finish(2 * i, slots[0])

                @pl.when(w2 < nwin)
                def _():
                    start(w2, slots[0])

                finish(2 * i + 1, slots[1])
                return carry

            lax.fori_loop(0, nwin // 2, body, 0)
            if nwin % 2:
                finish(nwin - 1, slots[0])

        for t in range(nt):
            run(tabs[t], outs[t], src_hbm if sels[t] else dst_hbm)

    return k


def kernel(s, v, edge_attr, edge_vec,
           s_proj_w1, s_proj_b1, s_proj_w2, s_proj_b2,
           edge_proj_w1, edge_proj_b1, edge_proj_w2, edge_proj_b2,
           final_s_w, final_s_b, final_v_w,
           U_w, V_w,
           s_mlp_w1, s_mlp_b1, s_mlp_w2, s_mlp_b2,
           edge_index):
    n, h = s.shape
    e = edge_attr.shape[0]
    src = edge_index[0]
    dst = edge_index[1]

    w1a_t = s_proj_w1[:, :h].T
    w1b_t = s_proj_w1[:, h:].T
    v_t = jnp.transpose(v, (2, 0, 1))
    a_tab, tb2_tab, t01_tab = _node_pre(
        s, w1a_t, w1b_t, v_t[0], v_t[1], v_t[2], bn=1000)

    prea, tb2, t01 = _make_sc_gather_tabs(
        e, h, w=400, sels=(0, 1, 1))(a_tab, tb2_tab, t01_tab, dst, src)

    ds_msg, dvm0, dvm1, dvm2 = _edge_stage(
        prea, tb2, t01, edge_attr, edge_vec,
        s_proj_b1.reshape(1, h), s_proj_w2.T.astype(jnp.bfloat16),
        s_proj_b2.reshape(1, 3 * h),
        edge_proj_w1.T, edge_proj_b1.reshape(1, h),
        edge_proj_w2.T.astype(jnp.bfloat16),
        edge_proj_b2.reshape(1, 3 * h), be=1000)

    scatter_pair = _make_sc_scatter_pair(e, n, h, w=160)
    zeros_nh = jnp.zeros((n, h), jnp.float32)
    ds, dv0 = scatter_pair(ds_msg, dvm0, dst, zeros_nh)
    dv1, dv2 = scatter_pair(dvm1, dvm2, dst, zeros_nh)

    m1a_t = s_mlp_w1[:, :h].T
    m1b_t = s_mlp_w1[:, h:].T
    s_out, vo0, vo1, vo2 = _node_update(
        s, v_t[0], v_t[1], v_t[2], ds, dv0, dv1, dv2,
        final_s_w.T, final_s_b.reshape(1, h), final_v_w.T,
        U_w.T, V_w.T, m1a_t, m1b_t, s_mlp_b1.reshape(1, h),
        s_mlp_w2.T, s_mlp_b2.reshape(1, 3 * h), bn=1000)

    v_out = jnp.stack([vo0, vo1, vo2], axis=-1)
    return (s_out, v_out)


# speedup vs baseline: 27.3295x; 1.0890x over previous
import functools

import jax
import jax.numpy as jnp
from jax import lax
from jax.experimental import pallas as pl
from jax.experimental.pallas import tpu as pltpu
from jax.experimental.pallas import tpu_sc as plsc

RES = 0.1
CLAMP = 100.0


def _silu(x):
    return x * jax.nn.sigmoid(x)


def _pack2(lo, hi):
    ul = jax.lax.bitcast_convert_type(
        lo.astype(jnp.bfloat16), jnp.uint16).astype(jnp.uint32)
    uh = jax.lax.bitcast_convert_type(
        hi.astype(jnp.bfloat16), jnp.uint16).astype(jnp.uint32)
    return jax.lax.bitcast_convert_type((uh << 16) | ul, jnp.float32)


def _unpack2(x):
    u = jax.lax.bitcast_convert_type(x, jnp.uint32)
    lo = jax.lax.bitcast_convert_type(u << 16, jnp.float32)
    hi = jax.lax.bitcast_convert_type(u & jnp.uint32(0xFFFF0000), jnp.float32)
    return lo, hi


def _node_pre_body(s_ref, w1a_ref, w1b_ref, v0_ref, v1_ref, v2_ref,
                   a_ref, tb2_ref, t01_ref):
    s = s_ref[...]
    a_ref[...] = jnp.dot(s, w1a_ref[...], preferred_element_type=jnp.float32)
    b = jnp.dot(s, w1b_ref[...], preferred_element_type=jnp.float32)
    tb2_ref[...] = _pack2(b, v2_ref[...])
    t01_ref[...] = _pack2(v0_ref[...], v1_ref[...])


def _node_pre(s, w1a_t, w1b_t, v0, v1, v2, bn):
    n, h = s.shape
    grid = (n // bn,)
    nb = lambda i: (i, 0)
    return pl.pallas_call(
        _node_pre_body,
        grid=grid,
        in_specs=[
            pl.BlockSpec((bn, h), nb),
            pl.BlockSpec((h, h), lambda i: (0, 0)),
            pl.BlockSpec((h, h), lambda i: (0, 0)),
            pl.BlockSpec((bn, h), nb),
            pl.BlockSpec((bn, h), nb),
            pl.BlockSpec((bn, h), nb),
        ],
        out_specs=[pl.BlockSpec((bn, h), nb)] * 3,
        out_shape=[jax.ShapeDtypeStruct((n, h), jnp.float32)] * 3,
    )(s, w1a_t, w1b_t, v0, v1, v2)


def _edge_body(prea_ref, tb2_ref, t01_ref, ea_ref, ev_ref,
               b1_ref, w2t_ref, b2_ref,
               ew1t_ref, eb1_ref, ew2t_ref, eb2_ref,
               ds_ref, dv0_ref, dv1_ref, dv2_ref):
    h = prea_ref.shape[1]
    pre_b, vj2 = _unpack2(tb2_ref[...])
    vj0, vj1 = _unpack2(t01_ref[...])
    h1 = _silu(prea_ref[...] + pre_b + b1_ref[...])
    phis = jnp.dot(h1.astype(jnp.bfloat16), w2t_ref[...],
                   preferred_element_type=jnp.float32) + b2_ref[...]
    he = _silu(jnp.dot(ea_ref[...], ew1t_ref[...],
                       preferred_element_type=jnp.float32) + eb1_ref[...])
    phie = jnp.dot(he.astype(jnp.bfloat16), ew2t_ref[...],
                   preferred_element_type=jnp.float32) + eb2_ref[...]
    phi = phis * phie
    ev = ev_ref[...]
    ds_ref[...] = phi[:, :h]
    phi2 = phi[:, h:2 * h]
    phi3 = phi[:, 2 * h:]
    dv0_ref[...] = phi2 * vj0 + phi3 * ev[:, 0:1]
    dv1_ref[...] = phi2 * vj1 + phi3 * ev[:, 1:2]
    dv2_ref[...] = phi2 * vj2 + phi3 * ev[:, 2:3]


def _edge_stage(prea, tb2, t01, ea, ev,
                b1, w2t, b2, ew1t, eb1, ew2t, eb2, be):
    e, h = prea.shape
    ed = ea.shape[1]
    grid = (e // be,)
    edge_block = lambda i: (i, 0)
    full = lambda i: (0, 0)
    return pl.pallas_call(
        _edge_body,
        grid=grid,
        in_specs=[
            pl.BlockSpec((be, h), edge_block),
            pl.BlockSpec((be, h), edge_block),
            pl.BlockSpec((be, h), edge_block),
            pl.BlockSpec((be, ed), edge_block),
            pl.BlockSpec((be, 3), edge_block),
            pl.BlockSpec((1, h), full),
            pl.BlockSpec((h, 3 * h), full),
            pl.BlockSpec((1, 3 * h), full),
            pl.BlockSpec((ed, h), full),
            pl.BlockSpec((1, h), full),
            pl.BlockSpec((h, 3 * h), full),
            pl.BlockSpec((1, 3 * h), full),
        ],
        out_specs=[pl.BlockSpec((be, h), edge_block)] * 4,
        out_shape=[jax.ShapeDtypeStruct((e, h), jnp.float32)] * 4,
    )(prea, tb2, t01, ea, ev, b1, w2t, b2, ew1t, eb1, ew2t, eb2)


def _node_update_body(s_ref, v0_ref, v1_ref, v2_ref,
                      dsa_ref, dv0a_ref, dv1a_ref, dv2a_ref,
                      dsb_ref, dv0b_ref, dv1b_ref, dv2b_ref,
                      fst_ref, fsb_ref, fvt_ref,
                      ut_ref, vt_ref,
                      m1at_ref, m1bt_ref, m1b_ref, m2t_ref, m2b_ref,
                      so_ref, vo0_ref, vo1_ref, vo2_ref):
    h = s_ref.shape[1]
    ds_m = dsa_ref[...] + dsb_ref[...]
    dv0_m = dv0a_ref[...] + dv0b_ref[...]
    dv1_m = dv1a_ref[...] + dv1b_ref[...]
    dv2_m = dv2a_ref[...] + dv2b_ref[...]
    ds = jnp.dot(ds_m, fst_ref[...], preferred_element_type=jnp.float32) + fsb_ref[...]
    s1 = s_ref[...] + RES * ds
    fvt = fvt_ref[...]
    v0 = v0_ref[...] + RES * jnp.dot(dv0_m, fvt, preferred_element_type=jnp.float32)
    v1 = v1_ref[...] + RES * jnp.dot(dv1_m, fvt, preferred_element_type=jnp.float32)
    v2 = v2_ref[...] + RES * jnp.dot(dv2_m, fvt, preferred_element_type=jnp.float32)
    ut = ut_ref[...]
    vt = vt_ref[...]
    uv0 = jnp.dot(v0, ut, preferred_element_type=jnp.float32)
    uv1 = jnp.dot(v1, ut, preferred_element_type=jnp.float32)
    uv2 = jnp.dot(v2, ut, preferred_element_type=jnp.float32)
    vv0 = jnp.dot(v0, vt, preferred_element_type=jnp.float32)
    vv1 = jnp.dot(v1, vt, preferred_element_type=jnp.float32)
    vv2 = jnp.dot(v2, vt, preferred_element_type=jnp.float32)
    vnorm = jnp.sqrt(vv0 * vv0 + vv1 * vv1 + vv2 * vv2)
    m1 = (jnp.dot(s1, m1at_ref[...], preferred_element_type=jnp.float32)
          + jnp.dot(vnorm, m1bt_ref[...], preferred_element_type=jnp.float32)
          + m1b_ref[...])
    a = jnp.dot(_silu(m1), m2t_ref[...], preferred_element_type=jnp.float32) + m2b_ref[...]
    a1 = a[:, :h]
    a2 = a[:, h:2 * h]
    a3 = a[:, 2 * h:]
    so_ref[...] = jnp.clip(s1 + RES * (a1 + a2 * vnorm), -CLAMP, CLAMP)
    vo0_ref[...] = jnp.clip(v0 + RES * (a3 * uv0), -CLAMP, CLAMP)
    vo1_ref[...] = jnp.clip(v1 + RES * (a3 * uv1), -CLAMP, CLAMP)
    vo2_ref[...] = jnp.clip(v2 + RES * (a3 * uv2), -CLAMP, CLAMP)


def _node_update(s, v0, v1, v2, msgs,
                 fst, fsb, fvt, ut, vt, m1at, m1bt, m1b, m2t, m2b, bn):
    n, h = s.shape
    grid = (n // bn,)
    nb = lambda i: (i, 0)
    full = lambda i: (0, 0)
    return pl.pallas_call(
        _node_update_body,
        grid=grid,
        in_specs=[pl.BlockSpec((bn, h), nb)] * 12 + [
            pl.BlockSpec((h, h), full),
            pl.BlockSpec((1, h), full),
            pl.BlockSpec((h, h), full),
            pl.BlockSpec((h, h), full),
            pl.BlockSpec((h, h), full),
            pl.BlockSpec((h, h), full),
            pl.BlockSpec((h, h), full),
            pl.BlockSpec((1, h), full),
            pl.BlockSpec((h, 3 * h), full),
            pl.BlockSpec((1, 3 * h), full),
        ],
        out_specs=[pl.BlockSpec((bn, h), nb)] * 4,
        out_shape=[jax.ShapeDtypeStruct((n, h), jnp.float32)] * 4,
    )(s, v0, v1, v2, *msgs,
      fst, fsb, fvt, ut, vt, m1at, m1bt, m1b, m2t, m2b)


def _make_sc_scatter_pair(e, n, h, w):
    mesh = plsc.VectorSubcoreMesh(core_axis_name="c", subcore_axis_name="s")
    nsub = 16
    rpt = (n // nsub) // 8 * 8
    tail = n - nsub * rpt
    edges_per_tile = e // nsub
    nwin = edges_per_tile // w

    @functools.partial(
        pl.kernel,
        mesh=mesh,
        out_type=[jax.ShapeDtypeStruct((n, h), jnp.float32)] * 2,
        scratch_types=[
            pltpu.VMEM_SHARED((n, h), jnp.float32),
            pltpu.VMEM((w,), jnp.int32),
            pltpu.VMEM((w,), jnp.int32),
            pltpu.VMEM((w, h), jnp.float32),
            pltpu.VMEM((w, h), jnp.float32),
            pltpu.SemaphoreType.DMA,
            pltpu.SemaphoreType.DMA,
            pltpu.SemaphoreType.DMA,
            pltpu.SemaphoreType.DMA,
        ],
    )
    def k(upd_a, upd_b, dst_hbm, zeros_hbm, out_a, out_b, acc,
          idx0, idx1, buf0, buf1, si0, si1, su0, su1):
        cid = lax.axis_index("c")
        sid = lax.axis_index("s")
        r0 = sid * rpt

        def rows_copy(src, dst_ref):
            pltpu.sync_copy(src.at[pl.ds(r0, rpt)], dst_ref.at[pl.ds(r0, rpt)])
            if tail:
                @pl.when(sid == 0)
                def _():
                    pltpu.sync_copy(src.at[pl.ds(nsub * rpt, tail)],
                                    dst_ref.at[pl.ds(nsub * rpt, tail)])

        rows_copy(zeros_hbm, acc)
        plsc.subcore_barrier()

        def run(upd_hbm):
            slots = ((idx0, buf0, si0, su0), (idx1, buf1, si1, su1))

            def start(wi, slot):
                idx_v, buf_v, sem_i, sem_u = slot
                base = sid * edges_per_tile + wi * w
                pltpu.async_copy(dst_hbm.at[pl.ds(base, w)], idx_v, sem_i)
                pltpu.async_copy(upd_hbm.at[pl.ds(base, w)], buf_v, sem_u)

            def finish(slot):
                idx_v, buf_v, sem_i, sem_u = slot
                pltpu.make_async_copy(dst_hbm.at[pl.ds(0, w)], idx_v, sem_i).wait()
                pltpu.make_async_copy(upd_hbm.at[pl.ds(0, w)], buf_v, sem_u).wait()
                pltpu.sync_copy(buf_v, acc.at[idx_v], add=True)

            start(0, slots[0])

            def body(i, carry):
                w2 = 2 * i + 2
                start(2 * i + 1, slots[1])
                finish(slots[0])

                @pl.when(w2 < nwin)
                def _():
                    start(w2, slots[0])

                finish(slots[1])
                return carry

            lax.fori_loop(0, nwin // 2, body, 0)
            if nwin % 2:
                finish(slots[0])

        @pl.when(cid == 0)
        def _():
            run(upd_a)

        @pl.when(cid == 1)
        def _():
            run(upd_b)

        plsc.subcore_barrier()

        @pl.when(cid == 0)
        def _():
            rows_copy(acc, out_a)

        @pl.when(cid == 1)
        def _():
            rows_copy(acc, out_b)

    return k


def _make_sc_gather_tabs(e, h, w, sels, dtype=jnp.float32):
    mesh = plsc.VectorSubcoreMesh(core_axis_name="c", subcore_axis_name="s")
    nworkers = 32
    epw = e // nworkers
    nwin = epw // w
    nt = len(sels)

    @functools.partial(
        pl.kernel,
        mesh=mesh,
        out_type=[jax.ShapeDtypeStruct((e, h), dtype)] * nt,
        scratch_types=[
            pltpu.VMEM((w,), jnp.int32),
            pltpu.VMEM((w, h), dtype),
            pltpu.VMEM((w,), jnp.int32),
            pltpu.VMEM((w, h), dtype),
            pltpu.SemaphoreType.DMA,
            pltpu.SemaphoreType.DMA,
        ],
    )
    def k(*refs):
        tabs = refs[:nt]
        dst_hbm, src_hbm = refs[nt], refs[nt + 1]
        outs = refs[nt + 2:nt + 2 + nt]
        idx0, buf0, idx1, buf1, sem0, sem1 = refs[nt + 2 + nt:]
        cid = lax.axis_index("c")
        sid = lax.axis_index("s")
        wid = sid * 2 + cid
        slots = ((idx0, buf0, sem0), (idx1, buf1, sem1))

        def run(tab, out, idx_hbm):
            def start(wi, slot):
                idx_v, buf, sem = slot
                base = wid * epw + wi * w
                pltpu.sync_copy(idx_hbm.at[pl.ds(base, w)], idx_v)
                pltpu.async_copy(tab.at[idx_v], buf, sem)

            def finish(wi, slot):
                idx_v, buf, sem = slot
                base = wid * epw + wi * w
                pltpu.make_async_copy(tab.at[idx_v], buf, sem).wait()
                pltpu.sync_copy(buf, out.at[pl.ds(base, w)])

            start(0, slots[0])

            def body(i, carry):
                w2 = 2 * i + 2
                start(2 * i + 1, slots[1])
                finish(2 * i, slots[0])

                @pl.when(w2 < nwin)
                def _():
                    start(w2, slots[0])

                finish(2 * i + 1, slots[1])
                return carry

            lax.fori_loop(0, nwin // 2, body, 0)
            if nwin % 2:
                finish(nwin - 1, slots[0])

        for t in range(nt):
            run(tabs[t], outs[t], src_hbm if sels[t] else dst_hbm)

    return k


def kernel(s, v, edge_attr, edge_vec,
           s_proj_w1, s_proj_b1, s_proj_w2, s_proj_b2,
           edge_proj_w1, edge_proj_b1, edge_proj_w2, edge_proj_b2,
           final_s_w, final_s_b, final_v_w,
           U_w, V_w,
           s_mlp_w1, s_mlp_b1, s_mlp_w2, s_mlp_b2,
           edge_index):
    n, h = s.shape
    e = edge_attr.shape[0]
    src = edge_index[0]
    dst = edge_index[1]

    w1a_t = s_proj_w1[:, :h].T
    w1b_t = s_proj_w1[:, h:].T
    v_t = jnp.transpose(v, (2, 0, 1))
    a_tab, tb2_tab, t01_tab = _node_pre(
        s, w1a_t, w1b_t, v_t[0], v_t[1], v_t[2], bn=1000)

    e0 = 153600
    chunks = ((0, e0), (e0, e - e0))
    zeros_nh = jnp.zeros((n, h), jnp.float32)
    msgs = []
    for off, ce in chunks:
        dst_c = lax.dynamic_slice_in_dim(dst, off, ce)
        src_c = lax.dynamic_slice_in_dim(src, off, ce)
        prea, tb2, t01 = _make_sc_gather_tabs(
            ce, h, w=400, sels=(0, 1, 1))(a_tab, tb2_tab, t01_tab, dst_c, src_c)

        ds_msg, dvm0, dvm1, dvm2 = _edge_stage(
            prea, tb2, t01,
            lax.dynamic_slice_in_dim(edge_attr, off, ce),
            lax.dynamic_slice_in_dim(edge_vec, off, ce),
            s_proj_b1.reshape(1, h), s_proj_w2.T.astype(jnp.bfloat16),
            s_proj_b2.reshape(1, 3 * h),
            edge_proj_w1.T, edge_proj_b1.reshape(1, h),
            edge_proj_w2.T.astype(jnp.bfloat16),
            edge_proj_b2.reshape(1, 3 * h), be=1280)

        scatter_pair = _make_sc_scatter_pair(ce, n, h, w=160)
        ds, dv0 = scatter_pair(ds_msg, dvm0, dst_c, zeros_nh)
        dv1, dv2 = scatter_pair(dvm1, dvm2, dst_c, zeros_nh)
        msgs += [ds, dv0, dv1, dv2]

    m1a_t = s_mlp_w1[:, :h].T
    m1b_t = s_mlp_w1[:, h:].T
    s_out, vo0, vo1, vo2 = _node_update(
        s, v_t[0], v_t[1], v_t[2], msgs,
        final_s_w.T, final_s_b.reshape(1, h), final_v_w.T,
        U_w.T, V_w.T, m1a_t, m1b_t, s_mlp_b1.reshape(1, h),
        s_mlp_w2.T, s_mlp_b2.reshape(1, 3 * h), bn=1000)

    v_out = jnp.stack([vo0, vo1, vo2], axis=-1)
    return (s_out, v_out)
